# Initial kernel scaffold; baseline (speedup 1.0000x reference)
import jax
import jax.numpy as jnp
from jax.experimental import pallas as pl


def kernel(x):
    raise NotImplementedError("write your pallas kernel here")


# speedup vs baseline: 1.1514x; 1.1514x over previous
import jax
import jax.numpy as jnp
from jax import lax
from jax.experimental import pallas as pl
from jax.experimental.pallas import tpu as pltpu
from jax.experimental.pallas import tpu_sc as plsc

_B, _N = 4096, 16384
_L = 16
_NW = 32
_ROWS_PER_W = _B // _NW
_VECS = _N // _L


def _sc_body(x_hbm, out_hbm, buf):
    c = lax.axis_index("c")
    s = lax.axis_index("s")
    wid = s * 2 + c
    base = wid * _ROWS_PER_W

    def row_loop(r, _):
        row = base + r
        pltpu.sync_copy(x_hbm.at[row], buf)

        def vec_loop(i, carry):
            v = buf[pl.ds(i * _L, _L)]
            inc = plsc.cumsum(v)
            buf[pl.ds(i * _L, _L)] = inc - v + carry
            return carry + jnp.sum(v)

        lax.fori_loop(0, _VECS, vec_loop, jnp.float32(0.0))
        pltpu.sync_copy(buf, out_hbm.at[row])
        return 0

    lax.fori_loop(0, _ROWS_PER_W, row_loop, 0)


def kernel(x):
    mesh = plsc.VectorSubcoreMesh(core_axis_name="c", subcore_axis_name="s")
    f = pl.kernel(
        _sc_body,
        mesh=mesh,
        out_type=jax.ShapeDtypeStruct((_B, _N), jnp.float32),
        scratch_types=[pltpu.VMEM((_N,), jnp.float32)],
        compiler_params=pltpu.CompilerParams(needs_layout_passes=False),
    )
    return f(x)


# speedup vs baseline: 3.9716x; 3.4494x over previous
import jax
import jax.numpy as jnp
from jax import lax
from jax.experimental import pallas as pl
from jax.experimental.pallas import tpu as pltpu
from jax.experimental.pallas import tpu_sc as plsc

_B, _N = 4096, 16384
_L = 16
_NW = 32
_ROWS_PER_W = _B // _NW
_VECS = _N // _L


def _sc_body(x_hbm, out_hbm, buf):
    c = lax.axis_index("c")
    s = lax.axis_index("s")
    wid = s * 2 + c
    base = wid * _ROWS_PER_W

    def row_loop(r, _):
        row = base + r
        pltpu.sync_copy(x_hbm.at[row], buf)

        def vec_loop(i, carry):
            v = buf[pl.ds(i * _L, _L)]
            inc = plsc.cumsum(v)
            buf[pl.ds(i * _L, _L)] = inc - v + carry
            return carry + inc[_L - 1]

        lax.fori_loop(0, _VECS, vec_loop, jnp.float32(0.0), unroll=8)
        pltpu.sync_copy(buf, out_hbm.at[row])
        return 0

    lax.fori_loop(0, _ROWS_PER_W, row_loop, 0)


def kernel(x):
    mesh = plsc.VectorSubcoreMesh(core_axis_name="c", subcore_axis_name="s")
    f = pl.kernel(
        _sc_body,
        mesh=mesh,
        out_type=jax.ShapeDtypeStruct((_B, _N), jnp.float32),
        scratch_types=[pltpu.VMEM((_N,), jnp.float32)],
        compiler_params=pltpu.CompilerParams(needs_layout_passes=False),
    )
    return f(x)


# speedup vs baseline: 4.6880x; 1.1804x over previous
import jax
import jax.numpy as jnp
from jax import lax
from jax.experimental import pallas as pl
from jax.experimental.pallas import tpu as pltpu
from jax.experimental.pallas import tpu_sc as plsc

_B, _N = 4096, 16384
_L = 16
_NW = 32
_ROWS_PER_W = _B // _NW
_VECS = _N // _L
_NB = 4


def _sc_body(x_hbm, out_hbm, buf, *sems):
    sems_in, sems_out = sems[:_NB], sems[_NB:]
    c = lax.axis_index("c")
    s = lax.axis_index("s")
    wid = s * 2 + c
    base = wid * _ROWS_PER_W

    def in_copy(r, slot):
        return pltpu.make_async_copy(
            x_hbm.at[pl.ds(base + r, 1)], buf.at[pl.ds(slot, 1)],
            sems_in[slot])

    def out_copy(r, slot):
        return pltpu.make_async_copy(
            buf.at[pl.ds(slot, 1)], out_hbm.at[pl.ds(base + r, 1)],
            sems_out[slot])

    in_copy(0, 0).start()
    in_copy(1, 1).start()

    def scan_row(slot):
        def vec_loop(i, carry):
            v = buf[slot, pl.ds(i * _L, _L)]
            inc = plsc.cumsum(v)
            buf[slot, pl.ds(i * _L, _L)] = inc - v + carry
            return carry + inc[_L - 1]

        lax.fori_loop(0, _VECS, vec_loop, jnp.float32(0.0), unroll=8)

    def outer(kk, _):
        for b in range(_NB):
            r = kk * _NB + b
            pslot = (b + 2) % _NB

            if b < 2:
                @pl.when(kk > 0)
                def _():
                    out_copy(r - 2, pslot).wait()
                in_copy(r + 2, pslot).start()
            else:
                @pl.when(kk < (_ROWS_PER_W // _NB) - 1)
                def _():
                    out_copy(r - 2, pslot).wait()
                    in_copy(r + 2, pslot).start()

            in_copy(r, b).wait()
            scan_row(b)
            out_copy(r, b).start()
        return 0

    lax.fori_loop(0, _ROWS_PER_W // _NB, outer, 0)

    last = _ROWS_PER_W - _NB
    for b in range(_NB):
        out_copy(last + b, b).wait()


def kernel(x):
    mesh = plsc.VectorSubcoreMesh(core_axis_name="c", subcore_axis_name="s")
    f = pl.kernel(
        _sc_body,
        mesh=mesh,
        out_type=jax.ShapeDtypeStruct((_B, _N), jnp.float32),
        scratch_types=[pltpu.VMEM((_NB, _N), jnp.float32)]
        + [pltpu.SemaphoreType.DMA] * (2 * _NB),
        compiler_params=pltpu.CompilerParams(needs_layout_passes=False),
    )
    return f(x)


# speedup vs baseline: 5.8039x; 1.2380x over previous
import jax
import jax.numpy as jnp
from jax import lax
from jax.experimental import pallas as pl
from jax.experimental.pallas import tpu as pltpu
from jax.experimental.pallas import tpu_sc as plsc

_B, _N = 4096, 16384
_L = 16
_NW = 32
_ROWS_PER_W = _B // _NW
_VECS = _N // _L
_NB = 4


def _sc_body(x_hbm, out_hbm, buf, *sems):
    sems_in, sems_out = sems[:_NB], sems[_NB:]
    c = lax.axis_index("c")
    s = lax.axis_index("s")
    wid = s * 2 + c
    base = wid * _ROWS_PER_W

    def in_copy(r, slot):
        return pltpu.make_async_copy(
            x_hbm.at[pl.ds(base + r, 1)], buf.at[pl.ds(slot, 1)],
            sems_in[slot])

    def out_copy(r, slot):
        return pltpu.make_async_copy(
            buf.at[pl.ds(slot, 1)], out_hbm.at[pl.ds(base + r, 1)],
            sems_out[slot])

    in_copy(0, 0).start()
    in_copy(1, 1).start()

    def scan_row(slot):
        @plsc.parallel_loop(0, _VECS, carry=jnp.float32(0.0), unroll=8)
        def _(i, carry):
            v = buf[slot, pl.ds(i * _L, _L)]
            inc = plsc.cumsum(v)
            buf[slot, pl.ds(i * _L, _L)] = inc - v + carry
            return carry + inc[_L - 1]

    def outer(kk, _):
        for b in range(_NB):
            r = kk * _NB + b
            pslot = (b + 2) % _NB

            if b < 2:
                @pl.when(kk > 0)
                def _():
                    out_copy(r - 2, pslot).wait()
                in_copy(r + 2, pslot).start()
            else:
                @pl.when(kk < (_ROWS_PER_W // _NB) - 1)
                def _():
                    out_copy(r - 2, pslot).wait()
                    in_copy(r + 2, pslot).start()

            in_copy(r, b).wait()
            scan_row(b)
            out_copy(r, b).start()
        return 0

    lax.fori_loop(0, _ROWS_PER_W // _NB, outer, 0)

    last = _ROWS_PER_W - _NB
    for b in range(_NB):
        out_copy(last + b, b).wait()


def kernel(x):
    mesh = plsc.VectorSubcoreMesh(core_axis_name="c", subcore_axis_name="s")
    f = pl.kernel(
        _sc_body,
        mesh=mesh,
        out_type=jax.ShapeDtypeStruct((_B, _N), jnp.float32),
        scratch_types=[pltpu.VMEM((_NB, _N), jnp.float32)]
        + [pltpu.SemaphoreType.DMA] * (2 * _NB),
        compiler_params=pltpu.CompilerParams(needs_layout_passes=False),
    )
    return f(x)
